# baseline (speedup 1.0000x reference)
import functools
import jax
import jax.numpy as jnp
from jax import lax
from jax.experimental import pallas as pl
from jax.experimental.pallas import tpu as pltpu
from jax.experimental.pallas import tpu_sc as plsc

N = 50000
D_IN = 256
D_OUT = 256
N_STEPS = 1000
ROWS = 5000

NW = 32
B_PAD = 51200
B_PER_W = B_PAD // NW
CHUNK = 200


def _sc_gather(idx_hbm, table_hbm, out_hbm, idx_v, rows_v, sem):
    wid = lax.axis_index("s") * 2 + lax.axis_index("c")
    base = wid * B_PER_W

    def body(j, _):
        off = base + j * CHUNK
        pltpu.sync_copy(idx_hbm.at[pl.ds(off, CHUNK)], idx_v)
        pltpu.async_copy(table_hbm.at[idx_v], rows_v, sem).wait()
        pltpu.sync_copy(rows_v, out_hbm.at[pl.ds(off, CHUNK)])
        return _

    lax.fori_loop(0, B_PER_W // CHUNK, body, 0)


def _tc_kernel(x_ref, g_ref, wt_ref, b_ref, o_ref):
    acc = jnp.dot(x_ref[...].astype(jnp.bfloat16), wt_ref[...],
                  preferred_element_type=jnp.float32)
    acc = acc + b_ref[...]
    u = (g_ref[...] * acc) * jnp.float32(1.4426950408889634)
    m = jnp.maximum(u, 0.0)
    e = jnp.exp2(u - (m + m))
    o_ref[...] = (m + jnp.log2(1.0 + e)) * jnp.float32(0.6931471805599453)


def kernel(x, t, W, b, time_embed):
    idx = jnp.pad(t.astype(jnp.int32), (0, B_PAD - N))
    mesh = plsc.VectorSubcoreMesh(core_axis_name="c", subcore_axis_name="s")
    gather = functools.partial(
        pl.kernel,
        mesh=mesh,
        out_type=jax.ShapeDtypeStruct((B_PAD, D_OUT), jnp.float32),
        scratch_types=[
            pltpu.VMEM((CHUNK,), jnp.int32),
            pltpu.VMEM((CHUNK, D_OUT), jnp.float32),
            pltpu.SemaphoreType.DMA,
        ],
    )(_sc_gather)
    gamma = gather(idx, time_embed)

    wt = W.T.astype(jnp.bfloat16)
    b2 = b.reshape(1, D_OUT)
    grid = (N // ROWS,)
    return pl.pallas_call(
        _tc_kernel,
        grid=grid,
        in_specs=[
            pl.BlockSpec((ROWS, D_IN), lambda i: (i, 0)),
            pl.BlockSpec((ROWS, D_OUT), lambda i: (i, 0)),
            pl.BlockSpec((D_IN, D_OUT), lambda i: (0, 0)),
            pl.BlockSpec((1, D_OUT), lambda i: (0, 0)),
        ],
        out_specs=pl.BlockSpec((ROWS, D_OUT), lambda i: (i, 0)),
        out_shape=jax.ShapeDtypeStruct((N, D_OUT), jnp.float32),
        compiler_params=pltpu.CompilerParams(
            dimension_semantics=("arbitrary",),
        ),
    )(x, gamma, wt, b2)


# speedup vs baseline: 2.8967x; 2.8967x over previous
import jax
import jax.numpy as jnp
from jax.experimental import pallas as pl
from jax.experimental.pallas import tpu as pltpu

N = 50000
D_IN = 256
D_OUT = 256
N_STEPS = 1000
ROWS = 5000


def _fused_kernel(x_ref, t_ref, wt_ref, b_ref, emb_ref, o_ref):
    acc = jnp.dot(x_ref[...].astype(jnp.bfloat16), wt_ref[...],
                  preferred_element_type=jnp.float32)
    acc = acc + b_ref[...]
    idx = t_ref[0, 0, :].astype(jnp.int16)
    steps = jax.lax.broadcasted_iota(jnp.int16, (ROWS, N_STEPS), 1)
    onehot = jnp.where(steps == idx[:, None],
                       jnp.bfloat16(1.0), jnp.bfloat16(0.0))
    gamma = jnp.dot(onehot, emb_ref[...], preferred_element_type=jnp.float32)
    u = (gamma * acc) * jnp.float32(1.4426950408889634)
    m = jnp.maximum(u, 0.0)
    e = jnp.exp2(u - (m + m))
    o_ref[...] = (m + jnp.log2(1.0 + e)) * jnp.float32(0.6931471805599453)


def kernel(x, t, W, b, time_embed):
    t3 = t.astype(jnp.int32).reshape(N // ROWS, 1, ROWS)
    wt = W.T.astype(jnp.bfloat16)
    b2 = b.reshape(1, D_OUT)
    emb16 = time_embed.astype(jnp.bfloat16)
    grid = (N // ROWS,)
    return pl.pallas_call(
        _fused_kernel,
        grid=grid,
        in_specs=[
            pl.BlockSpec((ROWS, D_IN), lambda i: (i, 0)),
            pl.BlockSpec((1, 1, ROWS), lambda i: (i, 0, 0)),
            pl.BlockSpec((D_IN, D_OUT), lambda i: (0, 0)),
            pl.BlockSpec((1, D_OUT), lambda i: (0, 0)),
            pl.BlockSpec((N_STEPS, D_OUT), lambda i: (0, 0)),
        ],
        out_specs=pl.BlockSpec((ROWS, D_OUT), lambda i: (i, 0)),
        out_shape=jax.ShapeDtypeStruct((N, D_OUT), jnp.float32),
        compiler_params=pltpu.CompilerParams(
            dimension_semantics=("parallel",),
        ),
    )(x, t3, wt, b2, emb16)
